# Initial kernel scaffold; baseline (speedup 1.0000x reference)
import jax
import jax.numpy as jnp
from jax.experimental import pallas as pl


def kernel(x, params):
    raise NotImplementedError("write your pallas kernel here")


# speedup vs baseline: 2.0285x; 2.0285x over previous
import functools

import jax
import jax.numpy as jnp
from jax.experimental import pallas as pl

F32 = jnp.float32

B, N, C = 2, 1024, 768
T = B * N
NH, HD = 12, 64
H = 576
E, K = 23, 3
LN_EPS = 1e-5


def _gelu(x):
    return 0.5 * x * (1.0 + jax.lax.erf(x * (2.0 ** -0.5)))


def _ln(x, g, b):
    m = x.mean(-1, keepdims=True)
    v = ((x - m) ** 2).mean(-1, keepdims=True)
    return (x - m) * jax.lax.rsqrt(v + LN_EPS) * g + b


def _qkv_kernel(x_ref, g_ref, b_ref, w_ref, bias_ref, o_ref):
    xn = _ln(x_ref[...], g_ref[...], b_ref[...])
    o_ref[...] = jnp.dot(xn, w_ref[...], preferred_element_type=F32) + bias_ref[...]


def _run_qkv(xf, g, b, w, bias):
    bt = 256
    return pl.pallas_call(
        _qkv_kernel,
        grid=(T // bt,),
        in_specs=[
            pl.BlockSpec((bt, C), lambda t: (t, 0)),
            pl.BlockSpec((1, C), lambda t: (0, 0)),
            pl.BlockSpec((1, C), lambda t: (0, 0)),
            pl.BlockSpec((C, 3 * C), lambda t: (0, 0)),
            pl.BlockSpec((1, 3 * C), lambda t: (0, 0)),
        ],
        out_specs=pl.BlockSpec((bt, 3 * C), lambda t: (t, 0)),
        out_shape=jax.ShapeDtypeStruct((T, 3 * C), F32),
    )(xf, g, b, w, bias)


def _attn_kernel(q_ref, k_ref, v_ref, o_ref):
    scale = HD ** -0.5
    for i in range(2):
        q = q_ref[0, :, i * HD:(i + 1) * HD]
        k = k_ref[0, :, i * HD:(i + 1) * HD]
        v = v_ref[0, :, i * HD:(i + 1) * HD]
        s = jax.lax.dot_general(q, k, (((1,), (1,)), ((), ())),
                                preferred_element_type=F32) * scale
        s = s - jnp.max(s, axis=1, keepdims=True)
        p = jnp.exp(s)
        p = p / jnp.sum(p, axis=1, keepdims=True)
        o_ref[0, :, i * HD:(i + 1) * HD] = jnp.dot(p, v, preferred_element_type=F32)


def _run_attn(qkv):
    return pl.pallas_call(
        _attn_kernel,
        grid=(B, NH // 2),
        in_specs=[
            pl.BlockSpec((1, N, 2 * HD), lambda b, j: (b, 0, j)),
            pl.BlockSpec((1, N, 2 * HD), lambda b, j: (b, 0, NH // 2 + j)),
            pl.BlockSpec((1, N, 2 * HD), lambda b, j: (b, 0, NH + j)),
        ],
        out_specs=pl.BlockSpec((1, N, 2 * HD), lambda b, j: (b, 0, j)),
        out_shape=jax.ShapeDtypeStruct((B, N, C), F32),
    )(qkv, qkv, qkv)


def _proj_kernel(a_ref, x_ref, wp_ref, bp_ref, g2_ref, b2_ref, wg_ref, bg_ref,
                 ws1_ref, bs1_ref, ws2_ref, bs2_ref,
                 base_ref, xn_ref, gw_ref):
    proj = jnp.dot(a_ref[...], wp_ref[...], preferred_element_type=F32) + bp_ref[...]
    x1 = x_ref[...] + proj
    xn = _ln(x1, g2_ref[...], b2_ref[...])
    xn_ref[...] = xn
    gw_ref[...] = jax.nn.sigmoid(
        jnp.dot(xn, wg_ref[...], preferred_element_type=F32) + bg_ref[...])
    h = _gelu(jnp.dot(xn, ws1_ref[...], preferred_element_type=F32) + bs1_ref[...])
    base_ref[...] = x1 + jnp.dot(h, ws2_ref[...], preferred_element_type=F32) + bs2_ref[...]


def _run_proj(attnf, xf, wp, bp, g2, b2, wg, bg, ws1, bs1, ws2, bs2):
    bt = 256
    return pl.pallas_call(
        _proj_kernel,
        grid=(T // bt,),
        in_specs=[
            pl.BlockSpec((bt, C), lambda t: (t, 0)),
            pl.BlockSpec((bt, C), lambda t: (t, 0)),
            pl.BlockSpec((C, C), lambda t: (0, 0)),
            pl.BlockSpec((1, C), lambda t: (0, 0)),
            pl.BlockSpec((1, C), lambda t: (0, 0)),
            pl.BlockSpec((1, C), lambda t: (0, 0)),
            pl.BlockSpec((C, E), lambda t: (0, 0)),
            pl.BlockSpec((1, E), lambda t: (0, 0)),
            pl.BlockSpec((C, H), lambda t: (0, 0)),
            pl.BlockSpec((1, H), lambda t: (0, 0)),
            pl.BlockSpec((H, C), lambda t: (0, 0)),
            pl.BlockSpec((1, C), lambda t: (0, 0)),
        ],
        out_specs=[
            pl.BlockSpec((bt, C), lambda t: (t, 0)),
            pl.BlockSpec((bt, C), lambda t: (t, 0)),
            pl.BlockSpec((bt, E), lambda t: (t, 0)),
        ],
        out_shape=[
            jax.ShapeDtypeStruct((T, C), F32),
            jax.ShapeDtypeStruct((T, C), F32),
            jax.ShapeDtypeStruct((T, E), F32),
        ],
    )(attnf, xf, wp, bp, g2, b2, wg, bg, ws1, bs1, ws2, bs2)


def _route_kernel(gw_ref, cmb_ref, aux_ref):
    g = gw_ref[...]
    iota = jax.lax.broadcasted_iota(jnp.int32, (T, E), 1)
    gm = g
    combine = jnp.zeros((T, E), F32)
    wsum = jnp.zeros((T, 1), F32)
    counts = jnp.zeros((1, E), F32)
    for _ in range(K):
        m = jnp.max(gm, axis=1, keepdims=True)
        sel = gm == m
        idx = jnp.min(jnp.where(sel, iota, E), axis=1, keepdims=True)
        onehot = (iota == idx).astype(F32)
        combine = combine + onehot * m
        wsum = wsum + m
        counts = counts + jnp.sum(onehot, axis=0, keepdims=True)
        gm = jnp.where(iota == idx, -1e30, gm)
    cmb_ref[...] = combine / wsum
    p = jnp.mean(g / jnp.sum(g, axis=1, keepdims=True), axis=0, keepdims=True)
    fload = counts * (E / (K * T))
    aux_ref[...] = jnp.sum(p * fload, keepdims=True).reshape(1, 1)


def _run_route(gw):
    return pl.pallas_call(
        _route_kernel,
        grid=(1,),
        in_specs=[pl.BlockSpec((T, E), lambda i: (0, 0))],
        out_specs=[
            pl.BlockSpec((T, E), lambda i: (0, 0)),
            pl.BlockSpec((1, 1), lambda i: (0, 0)),
        ],
        out_shape=[
            jax.ShapeDtypeStruct((T, E), F32),
            jax.ShapeDtypeStruct((1, 1), F32),
        ],
    )(gw)


def _expert_kernel(xn_ref, base_ref, cmb_ref, w1_ref, b1_ref, w2_ref, b2_ref, o_ref):
    e = pl.program_id(1)
    xt = xn_ref[...]
    h = _gelu(jnp.dot(xt, w1_ref[0], preferred_element_type=F32) + b1_ref[0])
    y = jnp.dot(h, w2_ref[0], preferred_element_type=F32) + b2_ref[0]
    cmb = cmb_ref[...]
    sel = jax.lax.broadcasted_iota(jnp.int32, cmb.shape, 1) == e
    w = jnp.sum(jnp.where(sel, cmb, 0.0), axis=1, keepdims=True)

    @pl.when(e == 0)
    def _():
        o_ref[...] = base_ref[...] + y * w

    @pl.when(e > 0)
    def _():
        o_ref[...] += y * w


def _run_experts(xn, base, cmb, we1, be1, we2, be2):
    bt = 512
    return pl.pallas_call(
        _expert_kernel,
        grid=(T // bt, E),
        in_specs=[
            pl.BlockSpec((bt, C), lambda t, e: (t, 0)),
            pl.BlockSpec((bt, C), lambda t, e: (t, 0)),
            pl.BlockSpec((bt, E), lambda t, e: (t, 0)),
            pl.BlockSpec((1, C, H), lambda t, e: (e, 0, 0)),
            pl.BlockSpec((1, 1, H), lambda t, e: (e, 0, 0)),
            pl.BlockSpec((1, H, C), lambda t, e: (e, 0, 0)),
            pl.BlockSpec((1, 1, C), lambda t, e: (e, 0, 0)),
        ],
        out_specs=pl.BlockSpec((bt, C), lambda t, e: (t, 0)),
        out_shape=jax.ShapeDtypeStruct((T, C), F32),
    )(xn, base, cmb, we1, be1, we2, be2)


def kernel(x, params):
    p = params
    xf = x.reshape(T, C)
    r2 = lambda a: a.reshape(1, -1)

    qkv = _run_qkv(xf, r2(p['ln1_g']), r2(p['ln1_b']), p['wqkv'], r2(p['bqkv']))
    attn = _run_attn(qkv.reshape(B, N, 3 * C))
    base, xn, gw = _run_proj(
        attn.reshape(T, C), xf, p['wproj'], r2(p['bproj']),
        r2(p['ln2_g']), r2(p['ln2_b']), p['wg'], r2(p['bg']),
        p['ws1'], r2(p['bs1']), p['ws2'], r2(p['bs2']))
    cmb, aux = _run_route(gw)
    out = _run_experts(xn, base, cmb,
                       p['we1'], p['be1'].reshape(E, 1, H),
                       p['we2'], p['be2'].reshape(E, 1, C))
    return out.reshape(B, N, C), aux.reshape(())


# speedup vs baseline: 2.0472x; 1.0092x over previous
import functools

import jax
import jax.numpy as jnp
from jax.experimental import pallas as pl

F32 = jnp.float32

B, N, C = 2, 1024, 768
T = B * N
NH, HD = 12, 64
H = 576
E, K = 23, 3
LN_EPS = 1e-5


BF16 = jnp.bfloat16


def _bdot(a, b):
    return jnp.dot(a.astype(BF16), b.astype(BF16), preferred_element_type=F32)


def _gelu(x):
    return 0.5 * x * (1.0 + jax.lax.erf(x * (2.0 ** -0.5)))


def _ln(x, g, b):
    m = x.mean(-1, keepdims=True)
    v = ((x - m) ** 2).mean(-1, keepdims=True)
    return (x - m) * jax.lax.rsqrt(v + LN_EPS) * g + b


def _qkv_kernel(x_ref, g_ref, b_ref, w_ref, bias_ref, o_ref):
    xn = _ln(x_ref[...], g_ref[...], b_ref[...])
    o_ref[...] = _bdot(xn, w_ref[...]) + bias_ref[...]


def _run_qkv(xf, g, b, w, bias):
    bt = 256
    return pl.pallas_call(
        _qkv_kernel,
        grid=(T // bt,),
        in_specs=[
            pl.BlockSpec((bt, C), lambda t: (t, 0)),
            pl.BlockSpec((1, C), lambda t: (0, 0)),
            pl.BlockSpec((1, C), lambda t: (0, 0)),
            pl.BlockSpec((C, 3 * C), lambda t: (0, 0)),
            pl.BlockSpec((1, 3 * C), lambda t: (0, 0)),
        ],
        out_specs=pl.BlockSpec((bt, 3 * C), lambda t: (t, 0)),
        out_shape=jax.ShapeDtypeStruct((T, 3 * C), F32),
    )(xf, g, b, w, bias)


def _attn_kernel(q_ref, k_ref, v_ref, o_ref):
    scale = HD ** -0.5
    for i in range(2):
        q = q_ref[0, :, i * HD:(i + 1) * HD]
        k = k_ref[0, :, i * HD:(i + 1) * HD]
        v = v_ref[0, :, i * HD:(i + 1) * HD]
        s = jax.lax.dot_general(q.astype(BF16), k.astype(BF16),
                                (((1,), (1,)), ((), ())),
                                preferred_element_type=F32) * scale
        s = s - jnp.max(s, axis=1, keepdims=True)
        p = jnp.exp(s)
        p = p / jnp.sum(p, axis=1, keepdims=True)
        o_ref[0, :, i * HD:(i + 1) * HD] = _bdot(p, v)


def _run_attn(qkv):
    return pl.pallas_call(
        _attn_kernel,
        grid=(B, NH // 2),
        in_specs=[
            pl.BlockSpec((1, N, 2 * HD), lambda b, j: (b, 0, j)),
            pl.BlockSpec((1, N, 2 * HD), lambda b, j: (b, 0, NH // 2 + j)),
            pl.BlockSpec((1, N, 2 * HD), lambda b, j: (b, 0, NH + j)),
        ],
        out_specs=pl.BlockSpec((1, N, 2 * HD), lambda b, j: (b, 0, j)),
        out_shape=jax.ShapeDtypeStruct((B, N, C), F32),
    )(qkv, qkv, qkv)


def _proj_kernel(a_ref, x_ref, wp_ref, bp_ref, g2_ref, b2_ref, wg_ref, bg_ref,
                 ws1_ref, bs1_ref, ws2_ref, bs2_ref,
                 base_ref, xn_ref, gw_ref):
    proj = _bdot(a_ref[...], wp_ref[...]) + bp_ref[...]
    x1 = x_ref[...] + proj
    xn = _ln(x1, g2_ref[...], b2_ref[...])
    xn_ref[...] = xn
    gw_ref[...] = jax.nn.sigmoid(
        jnp.dot(xn, wg_ref[...], preferred_element_type=F32) + bg_ref[...])
    h = _gelu(_bdot(xn, ws1_ref[...]) + bs1_ref[...])
    base_ref[...] = x1 + _bdot(h, ws2_ref[...]) + bs2_ref[...]


def _run_proj(attnf, xf, wp, bp, g2, b2, wg, bg, ws1, bs1, ws2, bs2):
    bt = 256
    return pl.pallas_call(
        _proj_kernel,
        grid=(T // bt,),
        in_specs=[
            pl.BlockSpec((bt, C), lambda t: (t, 0)),
            pl.BlockSpec((bt, C), lambda t: (t, 0)),
            pl.BlockSpec((C, C), lambda t: (0, 0)),
            pl.BlockSpec((1, C), lambda t: (0, 0)),
            pl.BlockSpec((1, C), lambda t: (0, 0)),
            pl.BlockSpec((1, C), lambda t: (0, 0)),
            pl.BlockSpec((C, E), lambda t: (0, 0)),
            pl.BlockSpec((1, E), lambda t: (0, 0)),
            pl.BlockSpec((C, H), lambda t: (0, 0)),
            pl.BlockSpec((1, H), lambda t: (0, 0)),
            pl.BlockSpec((H, C), lambda t: (0, 0)),
            pl.BlockSpec((1, C), lambda t: (0, 0)),
        ],
        out_specs=[
            pl.BlockSpec((bt, C), lambda t: (t, 0)),
            pl.BlockSpec((bt, C), lambda t: (t, 0)),
            pl.BlockSpec((bt, E), lambda t: (t, 0)),
        ],
        out_shape=[
            jax.ShapeDtypeStruct((T, C), F32),
            jax.ShapeDtypeStruct((T, C), F32),
            jax.ShapeDtypeStruct((T, E), F32),
        ],
    )(attnf, xf, wp, bp, g2, b2, wg, bg, ws1, bs1, ws2, bs2)


def _route_kernel(gw_ref, cmb_ref, aux_ref):
    g = gw_ref[...]
    iota = jax.lax.broadcasted_iota(jnp.int32, (T, E), 1)
    gm = g
    combine = jnp.zeros((T, E), F32)
    wsum = jnp.zeros((T, 1), F32)
    counts = jnp.zeros((1, E), F32)
    for _ in range(K):
        m = jnp.max(gm, axis=1, keepdims=True)
        sel = gm == m
        idx = jnp.min(jnp.where(sel, iota, E), axis=1, keepdims=True)
        onehot = (iota == idx).astype(F32)
        combine = combine + onehot * m
        wsum = wsum + m
        counts = counts + jnp.sum(onehot, axis=0, keepdims=True)
        gm = jnp.where(iota == idx, -1e30, gm)
    cmb_ref[...] = combine / wsum
    p = jnp.mean(g / jnp.sum(g, axis=1, keepdims=True), axis=0, keepdims=True)
    fload = counts * (E / (K * T))
    aux_ref[...] = jnp.sum(p * fload, keepdims=True).reshape(1, 1)


def _run_route(gw):
    return pl.pallas_call(
        _route_kernel,
        grid=(1,),
        in_specs=[pl.BlockSpec((T, E), lambda i: (0, 0))],
        out_specs=[
            pl.BlockSpec((T, E), lambda i: (0, 0)),
            pl.BlockSpec((1, 1), lambda i: (0, 0)),
        ],
        out_shape=[
            jax.ShapeDtypeStruct((T, E), F32),
            jax.ShapeDtypeStruct((1, 1), F32),
        ],
    )(gw)


def _expert_kernel(xn_ref, base_ref, cmb_ref, w1_ref, b1_ref, w2_ref, b2_ref, o_ref):
    e = pl.program_id(1)
    xt = xn_ref[...]
    h = _gelu(_bdot(xt, w1_ref[0]) + b1_ref[0])
    y = _bdot(h, w2_ref[0]) + b2_ref[0]
    cmb = cmb_ref[...]
    sel = jax.lax.broadcasted_iota(jnp.int32, cmb.shape, 1) == e
    w = jnp.sum(jnp.where(sel, cmb, 0.0), axis=1, keepdims=True)

    @pl.when(e == 0)
    def _():
        o_ref[...] = base_ref[...] + y * w

    @pl.when(e > 0)
    def _():
        o_ref[...] += y * w


def _run_experts(xn, base, cmb, we1, be1, we2, be2):
    bt = 512
    return pl.pallas_call(
        _expert_kernel,
        grid=(T // bt, E),
        in_specs=[
            pl.BlockSpec((bt, C), lambda t, e: (t, 0)),
            pl.BlockSpec((bt, C), lambda t, e: (t, 0)),
            pl.BlockSpec((bt, E), lambda t, e: (t, 0)),
            pl.BlockSpec((1, C, H), lambda t, e: (e, 0, 0)),
            pl.BlockSpec((1, 1, H), lambda t, e: (e, 0, 0)),
            pl.BlockSpec((1, H, C), lambda t, e: (e, 0, 0)),
            pl.BlockSpec((1, 1, C), lambda t, e: (e, 0, 0)),
        ],
        out_specs=pl.BlockSpec((bt, C), lambda t, e: (t, 0)),
        out_shape=jax.ShapeDtypeStruct((T, C), F32),
    )(xn, base, cmb, we1, be1, we2, be2)


def kernel(x, params):
    p = params
    xf = x.reshape(T, C)
    r2 = lambda a: a.reshape(1, -1)

    qkv = _run_qkv(xf, r2(p['ln1_g']), r2(p['ln1_b']), p['wqkv'], r2(p['bqkv']))
    attn = _run_attn(qkv.reshape(B, N, 3 * C))
    base, xn, gw = _run_proj(
        attn.reshape(T, C), xf, p['wproj'], r2(p['bproj']),
        r2(p['ln2_g']), r2(p['ln2_b']), p['wg'], r2(p['bg']),
        p['ws1'], r2(p['bs1']), p['ws2'], r2(p['bs2']))
    cmb, aux = _run_route(gw)
    out = _run_experts(xn, base, cmb,
                       p['we1'], p['be1'].reshape(E, 1, H),
                       p['we2'], p['be2'].reshape(E, 1, C))
    return out.reshape(B, N, C), aux.reshape(())


# speedup vs baseline: 2.2671x; 1.1074x over previous
import functools

import jax
import jax.numpy as jnp
from jax.experimental import pallas as pl

F32 = jnp.float32

B, N, C = 2, 1024, 768
T = B * N
NH, HD = 12, 64
H = 576
E, K = 23, 3
LN_EPS = 1e-5


BF16 = jnp.bfloat16


def _bdot(a, b):
    return jnp.dot(a.astype(BF16), b.astype(BF16), preferred_element_type=F32)


def _gelu(x):
    return 0.5 * x * (1.0 + jax.lax.erf(x * (2.0 ** -0.5)))


def _ln(x, g, b):
    m = x.mean(-1, keepdims=True)
    v = ((x - m) ** 2).mean(-1, keepdims=True)
    return (x - m) * jax.lax.rsqrt(v + LN_EPS) * g + b


def _qkv_kernel(x_ref, g_ref, b_ref, w_ref, bias_ref, o_ref):
    xn = _ln(x_ref[...], g_ref[...], b_ref[...])
    o_ref[...] = _bdot(xn, w_ref[...]) + bias_ref[...]


def _run_qkv(xf, g, b, w, bias):
    bt = 256
    return pl.pallas_call(
        _qkv_kernel,
        grid=(T // bt,),
        in_specs=[
            pl.BlockSpec((bt, C), lambda t: (t, 0)),
            pl.BlockSpec((1, C), lambda t: (0, 0)),
            pl.BlockSpec((1, C), lambda t: (0, 0)),
            pl.BlockSpec((C, 3 * C), lambda t: (0, 0)),
            pl.BlockSpec((1, 3 * C), lambda t: (0, 0)),
        ],
        out_specs=pl.BlockSpec((bt, 3 * C), lambda t: (t, 0)),
        out_shape=jax.ShapeDtypeStruct((T, 3 * C), F32),
    )(xf, g, b, w, bias)


def _attn_kernel(q_ref, k_ref, v_ref, o_ref):
    scale = HD ** -0.5
    for i in range(2):
        q = q_ref[0, :, i * HD:(i + 1) * HD]
        k = k_ref[0, :, i * HD:(i + 1) * HD]
        v = v_ref[0, :, i * HD:(i + 1) * HD]
        s = jax.lax.dot_general(q.astype(BF16), k.astype(BF16),
                                (((1,), (1,)), ((), ())),
                                preferred_element_type=F32) * scale
        s = s - jnp.max(s, axis=1, keepdims=True)
        p = jnp.exp(s)
        p = p / jnp.sum(p, axis=1, keepdims=True)
        o_ref[0, :, i * HD:(i + 1) * HD] = _bdot(p, v)


def _run_attn(qkv):
    return pl.pallas_call(
        _attn_kernel,
        grid=(B, NH // 2),
        in_specs=[
            pl.BlockSpec((1, N, 2 * HD), lambda b, j: (b, 0, j)),
            pl.BlockSpec((1, N, 2 * HD), lambda b, j: (b, 0, NH // 2 + j)),
            pl.BlockSpec((1, N, 2 * HD), lambda b, j: (b, 0, NH + j)),
        ],
        out_specs=pl.BlockSpec((1, N, 2 * HD), lambda b, j: (b, 0, j)),
        out_shape=jax.ShapeDtypeStruct((B, N, C), F32),
    )(qkv, qkv, qkv)


def _proj_kernel(a_ref, x_ref, wp_ref, bp_ref, g2_ref, b2_ref, wg_ref, bg_ref,
                 ws1_ref, bs1_ref, ws2_ref, bs2_ref,
                 base_ref, xn_ref, gw_ref):
    proj = _bdot(a_ref[...], wp_ref[...]) + bp_ref[...]
    x1 = x_ref[...] + proj
    xn = _ln(x1, g2_ref[...], b2_ref[...])
    xn_ref[...] = xn
    gw_ref[...] = jax.nn.sigmoid(
        jnp.dot(xn, wg_ref[...], preferred_element_type=F32) + bg_ref[...])
    h = _gelu(_bdot(xn, ws1_ref[...]) + bs1_ref[...])
    base_ref[...] = x1 + _bdot(h, ws2_ref[...]) + bs2_ref[...]


def _run_proj(attnf, xf, wp, bp, g2, b2, wg, bg, ws1, bs1, ws2, bs2):
    bt = 256
    return pl.pallas_call(
        _proj_kernel,
        grid=(T // bt,),
        in_specs=[
            pl.BlockSpec((bt, C), lambda t: (t, 0)),
            pl.BlockSpec((bt, C), lambda t: (t, 0)),
            pl.BlockSpec((C, C), lambda t: (0, 0)),
            pl.BlockSpec((1, C), lambda t: (0, 0)),
            pl.BlockSpec((1, C), lambda t: (0, 0)),
            pl.BlockSpec((1, C), lambda t: (0, 0)),
            pl.BlockSpec((C, E), lambda t: (0, 0)),
            pl.BlockSpec((1, E), lambda t: (0, 0)),
            pl.BlockSpec((C, H), lambda t: (0, 0)),
            pl.BlockSpec((1, H), lambda t: (0, 0)),
            pl.BlockSpec((H, C), lambda t: (0, 0)),
            pl.BlockSpec((1, C), lambda t: (0, 0)),
        ],
        out_specs=[
            pl.BlockSpec((bt, C), lambda t: (t, 0)),
            pl.BlockSpec((bt, C), lambda t: (t, 0)),
            pl.BlockSpec((bt, E), lambda t: (t, 0)),
        ],
        out_shape=[
            jax.ShapeDtypeStruct((T, C), F32),
            jax.ShapeDtypeStruct((T, C), F32),
            jax.ShapeDtypeStruct((T, E), F32),
        ],
    )(attnf, xf, wp, bp, g2, b2, wg, bg, ws1, bs1, ws2, bs2)


def _route_kernel(gw_ref, cmb_ref, aux_ref):
    g = gw_ref[...]
    iota = jax.lax.broadcasted_iota(jnp.int32, (T, E), 1)
    gm = g
    combine = jnp.zeros((T, E), F32)
    wsum = jnp.zeros((T, 1), F32)
    counts = jnp.zeros((1, E), F32)
    for _ in range(K):
        m = jnp.max(gm, axis=1, keepdims=True)
        sel = gm == m
        idx = jnp.min(jnp.where(sel, iota, E), axis=1, keepdims=True)
        onehot = (iota == idx).astype(F32)
        combine = combine + onehot * m
        wsum = wsum + m
        counts = counts + jnp.sum(onehot, axis=0, keepdims=True)
        gm = jnp.where(iota == idx, -1e30, gm)
    cmb_ref[...] = combine / wsum
    p = jnp.mean(g / jnp.sum(g, axis=1, keepdims=True), axis=0, keepdims=True)
    fload = counts * (E / (K * T))
    aux_ref[...] = jnp.sum(p * fload, keepdims=True).reshape(1, 1)


def _run_route(gw):
    return pl.pallas_call(
        _route_kernel,
        grid=(1,),
        in_specs=[pl.BlockSpec((T, E), lambda i: (0, 0))],
        out_specs=[
            pl.BlockSpec((T, E), lambda i: (0, 0)),
            pl.BlockSpec((1, 1), lambda i: (0, 0)),
        ],
        out_shape=[
            jax.ShapeDtypeStruct((T, E), F32),
            jax.ShapeDtypeStruct((1, 1), F32),
        ],
    )(gw)


def _expert_kernel(xn_ref, base_ref, cmb_ref, w1_ref, b1_ref, w2_ref, b2_ref, o_ref):
    e = pl.program_id(0)
    xt = xn_ref[...]
    h = _gelu(_bdot(xt, w1_ref[0]) + b1_ref[0])
    y = _bdot(h, w2_ref[0]) + b2_ref[0]
    cmb = cmb_ref[...]
    sel = jax.lax.broadcasted_iota(jnp.int32, cmb.shape, 1) == e
    w = jnp.sum(jnp.where(sel, cmb, 0.0), axis=1, keepdims=True)

    @pl.when(e == 0)
    def _():
        o_ref[...] = base_ref[...] + y * w

    @pl.when(e > 0)
    def _():
        o_ref[...] += y * w


def _run_experts(xn, base, cmb, we1, be1, we2, be2):
    return pl.pallas_call(
        _expert_kernel,
        grid=(E,),
        in_specs=[
            pl.BlockSpec((T, C), lambda e: (0, 0)),
            pl.BlockSpec((T, C), lambda e: (0, 0)),
            pl.BlockSpec((T, E), lambda e: (0, 0)),
            pl.BlockSpec((1, C, H), lambda e: (e, 0, 0)),
            pl.BlockSpec((1, 1, H), lambda e: (e, 0, 0)),
            pl.BlockSpec((1, H, C), lambda e: (e, 0, 0)),
            pl.BlockSpec((1, 1, C), lambda e: (e, 0, 0)),
        ],
        out_specs=pl.BlockSpec((T, C), lambda e: (0, 0)),
        out_shape=jax.ShapeDtypeStruct((T, C), F32),
    )(xn, base, cmb, we1, be1, we2, be2)


def kernel(x, params):
    p = params
    xf = x.reshape(T, C)
    r2 = lambda a: a.reshape(1, -1)

    qkv = _run_qkv(xf, r2(p['ln1_g']), r2(p['ln1_b']), p['wqkv'], r2(p['bqkv']))
    attn = _run_attn(qkv.reshape(B, N, 3 * C))
    base, xn, gw = _run_proj(
        attn.reshape(T, C), xf, p['wproj'], r2(p['bproj']),
        r2(p['ln2_g']), r2(p['ln2_b']), p['wg'], r2(p['bg']),
        p['ws1'], r2(p['bs1']), p['ws2'], r2(p['bs2']))
    cmb, aux = _run_route(gw)
    out = _run_experts(xn, base, cmb,
                       p['we1'], p['be1'].reshape(E, 1, H),
                       p['we2'], p['be2'].reshape(E, 1, C))
    return out.reshape(B, N, C), aux.reshape(())
